# baseline (speedup 1.0000x reference)
import jax
import jax.numpy as jnp
from jax.experimental import pallas as pl
from jax.experimental.pallas import tpu as pltpu

_B, _N, _S = 16, 64, 16
_GB = 16
_SB = _S + 2


def _leaky(z):
    return jnp.where(z >= 0, z, 0.05 * z)


def _dot(a, b):
    return jnp.dot(a, b, preferred_element_type=jnp.float32)


def _body(x_ref, a_ref, e_ref, W_pre_ref, b_pre_ref,
          V1_ref, b1_ref, V2_ref, b2_ref, V3_ref, b3_ref,
          W_d1_ref, b_d1_ref, W_d2_ref, b_d2_ref,
          W_zm_ref, b_zm_ref, W_zl_ref, b_zl_ref, eps_ref,
          out_ref, pooled_ref):
    g = pl.program_id(0)
    eye = jnp.eye(_N, dtype=jnp.float32)
    abs_ = [a_ref[b] for b in range(_GB)]
    masks = [x_ref[b][:, 32:33] for b in range(_GB)]
    aes = [jnp.concatenate(
        [abs_[b] * e_ref[b, s] for s in range(_S)] + [abs_[b], eye], axis=1)
        for b in range(_GB)]
    hs = [_leaky(_dot(x_ref[b][:, :32], W_pre_ref[...]) + b_pre_ref[...])
          for b in range(_GB)]

    def ecc(hs, V_ref, b_ref):
        P = _dot(jnp.concatenate(hs, axis=0), V_ref[...])
        out = []
        for b in range(_GB):
            Pb = P[b * _N:(b + 1) * _N]
            P2a = jnp.concatenate(
                [Pb[:, s * 32:(s + 1) * 32] for s in range(8)], axis=0)
            P2b = jnp.concatenate(
                [Pb[:, s * 32:(s + 1) * 32] for s in range(8, _SB)], axis=0)
            agg = _dot(aes[b][:, :512], P2a) + _dot(aes[b][:, 512:], P2b)
            out.append(_leaky(agg + b_ref[...]))
        return out

    hs = ecc(hs, V1_ref, b1_ref)
    hs = ecc(hs, V2_ref, b2_ref)
    hs = ecc(hs, V3_ref, b3_ref)
    pooled_rows = [jnp.sum(hs[b] * masks[b], axis=0, keepdims=True)
                   for b in range(_GB)]
    pooled_ref[pl.ds(g * _GB, _GB), :] = jnp.concatenate(pooled_rows, axis=0)

    @pl.when(g == _B // _GB - 1)
    def _head():
        p = pooled_ref[...]
        d1 = _leaky(_dot(p, W_d1_ref[...]) + b_d1_ref[...])
        d2 = _leaky(_dot(d1, W_d2_ref[...]) + b_d2_ref[...])
        zm = _dot(d2, W_zm_ref[...]) + b_zm_ref[...]
        zl = _dot(d2, W_zl_ref[...]) + b_zl_ref[...]
        out_ref[...] = zm + jnp.exp(0.5 * zl) * eps_ref[...]


@jax.jit
def _run(x, a, e_p, W_pre, b_pre, V1, b1, V2, b2, V3, b3,
         W_d1, b_d1, W_d2, b_d2, W_zm, b_zm, W_zl, b_zl, eps):
    step = lambda shp: pl.BlockSpec(shp, lambda i: (i,) + (0,) * (len(shp) - 1))
    full = lambda shp: pl.BlockSpec(shp, lambda i: (0,) * len(shp))
    in_specs = [
        step((_GB, _N, 33)),
        step((_GB, _N, _N)),
        step((_GB, _S, _N, _N)),
        full((32, 16)), full((1, 16)),
        full((16, _SB * 32)), full((1, 32)),
        full((32, _SB * 32)), full((1, 32)),
        full((32, _SB * 32)), full((1, 32)),
        full((32, 256)), full((1, 256)),
        full((256, 256)), full((1, 256)),
        full((256, 64)), full((1, 64)),
        full((256, 64)), full((1, 64)),
        full((_B, 64)),
    ]
    return pl.pallas_call(
        _body,
        grid=(_B // _GB,),
        in_specs=in_specs,
        out_specs=full((_B, 64)),
        out_shape=jax.ShapeDtypeStruct((_B, 64), jnp.float32),
        scratch_shapes=[pltpu.VMEM((_B, 32), jnp.float32)],
        compiler_params=pltpu.CompilerParams(
            allow_input_fusion=[True] * 20),
    )(x, a, e_p, W_pre, b_pre, V1, b1, V2, b2, V3, b3,
      W_d1, b_d1, W_d2, b_d2, W_zm, b_zm, W_zl, b_zl, eps)


def _vext(Wk, F, root):
    Vcat = Wk.reshape(_S, 32, F).transpose(2, 0, 1).reshape(F, _S * 32)
    return jnp.concatenate([Vcat, root], axis=1)


def kernel(x, a, e, W_pre, b_pre, Wk1, bk1, root1, b1, Wk2, bk2, root2, b2,
           Wk3, bk3, root3, b3, W_d1, b_d1, W_d2, b_d2, W_zm, b_zm,
           W_zl, b_zl, eps):
    e_p = e.transpose(0, 3, 1, 2)
    V1 = _vext(Wk1, 16, jnp.concatenate(
        [bk1.reshape(32, 16).T, root1], axis=1))
    V2 = _vext(Wk2, 32, jnp.concatenate(
        [bk2.reshape(32, 32).T, root2], axis=1))
    V3 = _vext(Wk3, 32, jnp.concatenate(
        [bk3.reshape(32, 32).T, root3], axis=1))
    row = lambda v: v.reshape(1, -1)
    return _run(x, a, e_p, W_pre, row(b_pre), V1, row(b1), V2, row(b2),
                V3, row(b3), W_d1, row(b_d1), W_d2, row(b_d2),
                W_zm, row(b_zm), W_zl, row(b_zl), eps)


# speedup vs baseline: 1.0141x; 1.0141x over previous
import jax
import jax.numpy as jnp
from jax.experimental import pallas as pl
from jax.experimental.pallas import tpu as pltpu

_B, _N, _S = 16, 64, 16
_GB = 8
_SB = _S + 2


def _leaky(z):
    return jnp.where(z >= 0, z, 0.05 * z)


def _dot(a, b):
    return jnp.dot(a, b, preferred_element_type=jnp.float32)


def _body(x_ref, a_ref, e_ref, W_pre_ref, b_pre_ref,
          V1_ref, b1_ref, V2_ref, b2_ref, V3_ref, b3_ref,
          W_d1_ref, b_d1_ref, W_d2_ref, b_d2_ref,
          W_zm_ref, b_zm_ref, W_zl_ref, b_zl_ref, eps_ref,
          out_ref, pooled_ref):
    g = pl.program_id(0)
    eye = jnp.eye(_N, dtype=jnp.float32)
    x2 = x_ref[...].reshape(_GB * _N, 33)
    abs_ = [a_ref[b] for b in range(_GB)]
    aes = [jnp.concatenate(
        [abs_[b] * e_ref[b, s] for s in range(_S)] + [abs_[b], eye], axis=1)
        for b in range(_GB)]
    H = _leaky(_dot(x2[:, :32], W_pre_ref[...]) + b_pre_ref[...])

    def ecc(H, V_ref, b_ref):
        P = _dot(H, V_ref[...])
        out = []
        for b in range(_GB):
            Pb = P[b * _N:(b + 1) * _N]
            P2a = jnp.concatenate(
                [Pb[:, s * 32:(s + 1) * 32] for s in range(8)], axis=0)
            P2b = jnp.concatenate(
                [Pb[:, s * 32:(s + 1) * 32] for s in range(8, _SB)], axis=0)
            out.append(_dot(aes[b][:, :512], P2a) + _dot(aes[b][:, 512:], P2b))
        return _leaky(jnp.concatenate(out, axis=0) + b_ref[...])

    H = ecc(H, V1_ref, b1_ref)
    H = ecc(H, V2_ref, b2_ref)
    H = ecc(H, V3_ref, b3_ref)
    pooled = jnp.sum((H * x2[:, 32:33]).reshape(_GB, _N, 32), axis=1)
    pooled_ref[pl.ds(g * _GB, _GB), :] = pooled

    @pl.when(g == _B // _GB - 1)
    def _head():
        p = pooled_ref[...]
        d1 = _leaky(_dot(p, W_d1_ref[...]) + b_d1_ref[...])
        d2 = _leaky(_dot(d1, W_d2_ref[...]) + b_d2_ref[...])
        zm = _dot(d2, W_zm_ref[...]) + b_zm_ref[...]
        zl = _dot(d2, W_zl_ref[...]) + b_zl_ref[...]
        out_ref[...] = zm + jnp.exp(0.5 * zl) * eps_ref[...]


@jax.jit
def _run(x, a, e_p, W_pre, b_pre, V1, b1, V2, b2, V3, b3,
         W_d1, b_d1, W_d2, b_d2, W_zm, b_zm, W_zl, b_zl, eps):
    step = lambda shp: pl.BlockSpec(shp, lambda i: (i,) + (0,) * (len(shp) - 1))
    full = lambda shp: pl.BlockSpec(shp, lambda i: (0,) * len(shp))
    in_specs = [
        step((_GB, _N, 33)),
        step((_GB, _N, _N)),
        step((_GB, _S, _N, _N)),
        full((32, 16)), full((1, 16)),
        full((16, _SB * 32)), full((1, 32)),
        full((32, _SB * 32)), full((1, 32)),
        full((32, _SB * 32)), full((1, 32)),
        full((32, 256)), full((1, 256)),
        full((256, 256)), full((1, 256)),
        full((256, 64)), full((1, 64)),
        full((256, 64)), full((1, 64)),
        full((_B, 64)),
    ]
    return pl.pallas_call(
        _body,
        grid=(_B // _GB,),
        in_specs=in_specs,
        out_specs=full((_B, 64)),
        out_shape=jax.ShapeDtypeStruct((_B, 64), jnp.float32),
        scratch_shapes=[pltpu.VMEM((_B, 32), jnp.float32)],
        compiler_params=pltpu.CompilerParams(
            allow_input_fusion=[True] * 20),
    )(x, a, e_p, W_pre, b_pre, V1, b1, V2, b2, V3, b3,
      W_d1, b_d1, W_d2, b_d2, W_zm, b_zm, W_zl, b_zl, eps)


def _vext(Wk, F, root):
    Vcat = Wk.reshape(_S, 32, F).transpose(2, 0, 1).reshape(F, _S * 32)
    return jnp.concatenate([Vcat, root], axis=1)


def kernel(x, a, e, W_pre, b_pre, Wk1, bk1, root1, b1, Wk2, bk2, root2, b2,
           Wk3, bk3, root3, b3, W_d1, b_d1, W_d2, b_d2, W_zm, b_zm,
           W_zl, b_zl, eps):
    e_p = e.transpose(0, 3, 1, 2)
    V1 = _vext(Wk1, 16, jnp.concatenate(
        [bk1.reshape(32, 16).T, root1], axis=1))
    V2 = _vext(Wk2, 32, jnp.concatenate(
        [bk2.reshape(32, 32).T, root2], axis=1))
    V3 = _vext(Wk3, 32, jnp.concatenate(
        [bk3.reshape(32, 32).T, root3], axis=1))
    row = lambda v: v.reshape(1, -1)
    return _run(x, a, e_p, W_pre, row(b_pre), V1, row(b1), V2, row(b2),
                V3, row(b3), W_d1, row(b_d1), W_d2, row(b_d2),
                W_zm, row(b_zm), W_zl, row(b_zl), eps)


# speedup vs baseline: 1.0259x; 1.0116x over previous
import jax
import jax.numpy as jnp
from jax.experimental import pallas as pl
from jax.experimental.pallas import tpu as pltpu

_B, _N, _S = 16, 64, 16
_GB = 8
_SB = _S + 2


def _leaky(z):
    return jnp.where(z >= 0, z, 0.05 * z)


def _dot(a, b):
    return jnp.dot(a, b, preferred_element_type=jnp.float32)


def _body(x_ref, a_ref, e_ref, W_pre_ref, b_pre_ref,
          V1_ref, b1_ref, V2_ref, b2_ref, V3_ref, b3_ref,
          W_d1_ref, b_d1_ref, W_d2_ref, b_d2_ref,
          W_zm_ref, b_zm_ref, W_zl_ref, b_zl_ref, eps_ref,
          out_ref, pooled_ref):
    g = pl.program_id(0)
    eye = jnp.eye(_N, dtype=jnp.float32)
    x2 = x_ref[...].reshape(_GB * _N, 33)
    abs_ = [a_ref[b] for b in range(_GB)]
    aes = [jnp.concatenate(
        [abs_[b] * e_ref[b, s] for s in range(_S)] + [abs_[b], eye], axis=1)
        for b in range(_GB)]
    H = _leaky(_dot(x2[:, :32], W_pre_ref[...]) + b_pre_ref[...])

    def ecc(H, V_ref, b_ref):
        P = _dot(H, V_ref[...])
        out = []
        for b in range(_GB):
            Pb = P[b * _N:(b + 1) * _N]
            P2 = jnp.concatenate(
                [Pb[:, s * 32:(s + 1) * 32] for s in range(_SB)], axis=0)
            out.append(_dot(aes[b], P2))
        return _leaky(jnp.concatenate(out, axis=0) + b_ref[...])

    H = ecc(H, V1_ref, b1_ref)
    H = ecc(H, V2_ref, b2_ref)
    H = ecc(H, V3_ref, b3_ref)
    pooled = jnp.sum((H * x2[:, 32:33]).reshape(_GB, _N, 32), axis=1)
    pooled_ref[pl.ds(g * _GB, _GB), :] = pooled

    @pl.when(g == _B // _GB - 1)
    def _head():
        p = pooled_ref[...]
        d1 = _leaky(_dot(p, W_d1_ref[...]) + b_d1_ref[...])
        d2 = _leaky(_dot(d1, W_d2_ref[...]) + b_d2_ref[...])
        zm = _dot(d2, W_zm_ref[...]) + b_zm_ref[...]
        zl = _dot(d2, W_zl_ref[...]) + b_zl_ref[...]
        out_ref[...] = zm + jnp.exp(0.5 * zl) * eps_ref[...]


@jax.jit
def _run(x, a, e_p, W_pre, b_pre, V1, b1, V2, b2, V3, b3,
         W_d1, b_d1, W_d2, b_d2, W_zm, b_zm, W_zl, b_zl, eps):
    step = lambda shp: pl.BlockSpec(shp, lambda i: (i,) + (0,) * (len(shp) - 1))
    full = lambda shp: pl.BlockSpec(shp, lambda i: (0,) * len(shp))
    in_specs = [
        step((_GB, _N, 33)),
        step((_GB, _N, _N)),
        step((_GB, _S, _N, _N)),
        full((32, 16)), full((1, 16)),
        full((16, _SB * 32)), full((1, 32)),
        full((32, _SB * 32)), full((1, 32)),
        full((32, _SB * 32)), full((1, 32)),
        full((32, 256)), full((1, 256)),
        full((256, 256)), full((1, 256)),
        full((256, 64)), full((1, 64)),
        full((256, 64)), full((1, 64)),
        full((_B, 64)),
    ]
    return pl.pallas_call(
        _body,
        grid=(_B // _GB,),
        in_specs=in_specs,
        out_specs=full((_B, 64)),
        out_shape=jax.ShapeDtypeStruct((_B, 64), jnp.float32),
        scratch_shapes=[pltpu.VMEM((_B, 32), jnp.float32)],
        compiler_params=pltpu.CompilerParams(
            allow_input_fusion=[True] * 20),
    )(x, a, e_p, W_pre, b_pre, V1, b1, V2, b2, V3, b3,
      W_d1, b_d1, W_d2, b_d2, W_zm, b_zm, W_zl, b_zl, eps)


def _vext(Wk, F, root):
    Vcat = Wk.reshape(_S, 32, F).transpose(2, 0, 1).reshape(F, _S * 32)
    return jnp.concatenate([Vcat, root], axis=1)


def kernel(x, a, e, W_pre, b_pre, Wk1, bk1, root1, b1, Wk2, bk2, root2, b2,
           Wk3, bk3, root3, b3, W_d1, b_d1, W_d2, b_d2, W_zm, b_zm,
           W_zl, b_zl, eps):
    e_p = e.transpose(0, 3, 1, 2)
    V1 = _vext(Wk1, 16, jnp.concatenate(
        [bk1.reshape(32, 16).T, root1], axis=1))
    V2 = _vext(Wk2, 32, jnp.concatenate(
        [bk2.reshape(32, 32).T, root2], axis=1))
    V3 = _vext(Wk3, 32, jnp.concatenate(
        [bk3.reshape(32, 32).T, root3], axis=1))
    row = lambda v: v.reshape(1, -1)
    return _run(x, a, e_p, W_pre, row(b_pre), V1, row(b1), V2, row(b2),
                V3, row(b3), W_d1, row(b_d1), W_d2, row(b_d2),
                W_zm, row(b_zm), W_zl, row(b_zl), eps)
